# Initial kernel scaffold; baseline (speedup 1.0000x reference)
import jax
import jax.numpy as jnp
from jax.experimental import pallas as pl


def kernel(xyz):
    raise NotImplementedError("write your pallas kernel here")


# speedup vs baseline: 4.5375x; 4.5375x over previous
import functools

import jax
import jax.numpy as jnp
from jax.experimental import pallas as pl

B = 8
N = 8192
G = 256
K = 32
_INF = float("inf")


def _group_kernel(x_ref, y_ref, z_ref,
                  nbx_ref, nby_ref, nbz_ref,
                  cx_ref, cy_ref, cz_ref):
    f32 = jnp.float32
    x = x_ref[:, :]
    y = y_ref[:, :]
    z = z_ref[:, :]

    iota_bn = jax.lax.broadcasted_iota(jnp.int32, (B, N), 1)
    iota_bg = jax.lax.broadcasted_iota(jnp.int32, (B, G), 1)

    def fps_body(i, carry):
        dmin, far, cxa, cya, cza = carry
        sel = iota_bn == far
        cxi = jnp.sum(jnp.where(sel, x, 0.0), axis=1, keepdims=True)
        cyi = jnp.sum(jnp.where(sel, y, 0.0), axis=1, keepdims=True)
        czi = jnp.sum(jnp.where(sel, z, 0.0), axis=1, keepdims=True)
        colm = iota_bg == i
        cxa = jnp.where(colm, cxi, cxa)
        cya = jnp.where(colm, cyi, cya)
        cza = jnp.where(colm, czi, cza)
        dx = x - cxi
        dy = y - cyi
        dz = z - czi
        d = (dx * dx + dy * dy) + dz * dz
        dmin = jnp.minimum(dmin, d)
        mx = jnp.max(dmin, axis=1, keepdims=True)
        far = jnp.min(jnp.where(dmin == mx, iota_bn, N), axis=1, keepdims=True)
        return dmin, far, cxa, cya, cza

    dmin0 = jnp.full((B, N), _INF, dtype=f32)
    far0 = jnp.zeros((B, 1), dtype=jnp.int32)
    ca0 = jnp.zeros((B, G), dtype=f32)
    _, _, cxa, cya, cza = jax.lax.fori_loop(
        0, G, fps_body, (dmin0, far0, ca0, ca0, ca0))

    iota_bgg1 = jax.lax.broadcasted_iota(jnp.int32, (B, G, G), 1)
    iota_bgg2 = jax.lax.broadcasted_iota(jnp.int32, (B, G, G), 2)
    iota_b3 = jax.lax.broadcasted_iota(jnp.int32, (B, G, G), 0)
    iota_b1 = jax.lax.broadcasted_iota(jnp.int32, (B, 1), 0)

    def build_d(b, dacc):
        sel = iota_b1 == b
        cxb = jnp.sum(jnp.where(sel, cxa, 0.0), axis=0, keepdims=True)
        cyb = jnp.sum(jnp.where(sel, cya, 0.0), axis=0, keepdims=True)
        czb = jnp.sum(jnp.where(sel, cza, 0.0), axis=0, keepdims=True)
        cb = jnp.concatenate(
            [jnp.transpose(cxb), jnp.transpose(cyb), jnp.transpose(czb)],
            axis=1)
        ab = jax.lax.dot_general(cb, cb, (((1,), (1,)), ((), ())),
                                 preferred_element_type=f32)
        a2 = jnp.sum(cb * cb, axis=1, keepdims=True)
        db = a2 + jnp.transpose(a2) - 2.0 * ab
        dacc = jnp.where(iota_b3 == b, db[None, :, :], dacc)
        return dacc

    dmat = jax.lax.fori_loop(0, B, build_d,
                             jnp.zeros((B, G, G), dtype=f32))
    dmat = jnp.where(iota_bgg1 == iota_bgg2, _INF, dmat)

    def chain_body(s, carry):
        visited, cur, orda = carry
        rowm = iota_bgg1 == cur[:, :, None]
        row = jnp.sum(jnp.where(rowm, dmat, 0.0), axis=1)
        row = jnp.where(visited != 0, _INF, row)
        mn = jnp.min(row, axis=1, keepdims=True)
        nxt = jnp.min(jnp.where(row == mn, iota_bg, G), axis=1, keepdims=True)
        orda = jnp.where(iota_bg == s, nxt, orda)
        visited = jnp.maximum(visited, (iota_bg == nxt).astype(jnp.int32))
        return visited, nxt, orda

    visited0 = (iota_bg == 0).astype(jnp.int32)
    order0 = jnp.zeros((B, G), dtype=jnp.int32)
    _, _, order = jax.lax.fori_loop(
        1, G, chain_body, (visited0, far0 * 0, order0))

    iota_gn = jax.lax.broadcasted_iota(jnp.int32, (G, N), 1)
    iota_gk = jax.lax.broadcasted_iota(jnp.int32, (G, K), 1)
    iota_gg2 = jax.lax.broadcasted_iota(jnp.int32, (G, G), 1)

    def knn_batch(b, _):
        sel = iota_b1 == b
        cxb = jnp.transpose(jnp.sum(jnp.where(sel, cxa, 0.0), axis=0,
                                    keepdims=True))
        cyb = jnp.transpose(jnp.sum(jnp.where(sel, cya, 0.0), axis=0,
                                    keepdims=True))
        czb = jnp.transpose(jnp.sum(jnp.where(sel, cza, 0.0), axis=0,
                                    keepdims=True))
        ordb = jnp.transpose(jnp.sum(jnp.where(sel, order, 0), axis=0,
                                     keepdims=True))
        xb = x_ref[pl.ds(b, 1), :]
        yb = y_ref[pl.ds(b, 1), :]
        zb = z_ref[pl.ds(b, 1), :]
        cb = jnp.concatenate([cxb, cyb, czb], axis=1)
        xm = jnp.concatenate([xb, yb, zb], axis=0)
        ab = jax.lax.dot_general(cb, xm, (((1,), (0,)), ((), ())),
                                 preferred_element_type=jnp.float32)
        a2 = jnp.sum(cb * cb, axis=1, keepdims=True)
        b2 = xb * xb + yb * yb + zb * zb
        d2 = a2 + b2 - 2.0 * ab

        def sel_body(k, carry):
            d2c, nx, ny, nz = carry
            mn = jnp.min(d2c, axis=1, keepdims=True)
            idx = jnp.min(jnp.where(d2c == mn, iota_gn, N), axis=1,
                          keepdims=True)
            oh = iota_gn == idx
            gx = jnp.sum(jnp.where(oh, xb, 0.0), axis=1, keepdims=True)
            gy = jnp.sum(jnp.where(oh, yb, 0.0), axis=1, keepdims=True)
            gz = jnp.sum(jnp.where(oh, zb, 0.0), axis=1, keepdims=True)
            colk = iota_gk == k
            nx = jnp.where(colk, gx, nx)
            ny = jnp.where(colk, gy, ny)
            nz = jnp.where(colk, gz, nz)
            d2c = jnp.where(oh, _INF, d2c)
            return d2c, nx, ny, nz

        nb0 = jnp.zeros((G, K), dtype=f32)
        _, nx, ny, nz = jax.lax.fori_loop(0, K, sel_body, (d2, nb0, nb0, nb0))
        nx = nx - cxb
        ny = ny - cyb
        nz = nz - czb

        pb = (ordb == iota_gg2).astype(f32)
        hi = jax.lax.Precision.HIGHEST
        dot = functools.partial(jax.lax.dot_general,
                                dimension_numbers=(((1,), (0,)), ((), ())),
                                preferred_element_type=f32, precision=hi)
        nbx_ref[pl.ds(b, 1)] = dot(pb, nx)[None]
        nby_ref[pl.ds(b, 1)] = dot(pb, ny)[None]
        nbz_ref[pl.ds(b, 1)] = dot(pb, nz)[None]
        cx_ref[pl.ds(b, 1)] = jnp.transpose(dot(pb, cxb))
        cy_ref[pl.ds(b, 1)] = jnp.transpose(dot(pb, cyb))
        cz_ref[pl.ds(b, 1)] = jnp.transpose(dot(pb, czb))
        return 0

    jax.lax.fori_loop(0, B, knn_batch, 0)


@jax.jit
def kernel(xyz):
    x = xyz[:, :, 0]
    y = xyz[:, :, 1]
    z = xyz[:, :, 2]
    f32 = jnp.float32
    outs = pl.pallas_call(
        _group_kernel,
        out_shape=(
            jax.ShapeDtypeStruct((B, G, K), f32),
            jax.ShapeDtypeStruct((B, G, K), f32),
            jax.ShapeDtypeStruct((B, G, K), f32),
            jax.ShapeDtypeStruct((B, G), f32),
            jax.ShapeDtypeStruct((B, G), f32),
            jax.ShapeDtypeStruct((B, G), f32),
        ),
    )(x, y, z)
    nbx, nby, nbz, cx, cy, cz = outs
    neighborhood = jnp.stack([nbx, nby, nbz], axis=-1)
    center = jnp.stack([cx, cy, cz], axis=-1)
    return neighborhood, center
